# baseline (device time: 73886 ns/iter reference)
import jax
import jax.numpy as jnp
from jax import lax
from jax.experimental import pallas as pl
from jax.experimental.pallas import tpu as pltpu

N_DEV = 4
CAPD = 320
CAPE = 192


def kernel(x, assign, W1, W2):
    T, D = x.shape
    E, _, F = W1.shape
    WFC = F // 2

    xb = x.astype(jnp.bfloat16)
    a2d = assign.reshape(T, 1)

    def body(x_ref, a_ref, w1_any, w2_any, out_ref,
             w1b, w2b, wstage, sel_store,
             xg_send, ag_send, xg_recv, ag_recv, ret_send, ret_recv,
             wsem, dx_s, dx_r, da_s, da_r, rt_s, rt_r):
        my = lax.axis_index("i")
        right = lax.rem(my + 1, N_DEV)
        left = lax.rem(my + N_DEV - 1, N_DEV)
        diag = lax.rem(my + 2, N_DEV)
        e0 = my * 2

        def tri(M):
            ri = lax.broadcasted_iota(jnp.int32, (M, M), 0)
            ci = lax.broadcasted_iota(jnp.int32, (M, M), 1)
            return (ci < ri).astype(jnp.bfloat16)

        def build_sel(mask, ls, cap):
            M = mask.shape[0]
            mf = mask.astype(jnp.bfloat16)
            rank = jnp.dot(
                ls, mf, preferred_element_type=jnp.float32
            ).astype(jnp.int32)
            cols = lax.broadcasted_iota(jnp.int32, (M, cap), 1)
            return jnp.where(
                jnp.logical_and(mask, rank == cols), 1.0, 0.0
            ).astype(jnp.bfloat16)

        def gather(sel, v):
            return lax.dot_general(
                sel, v, (((0,), (0,)), ((), ())),
                preferred_element_type=jnp.float32,
            )

        barrier = pltpu.get_barrier_semaphore()
        for nbr in (left, right, diag):
            pl.semaphore_signal(barrier, inc=1, device_id=(nbr,),
                                device_id_type=pl.DeviceIdType.MESH)
        pl.semaphore_wait(barrier, 3)

        ls_T = tri(T)
        ls_D = tri(CAPD)
        a_all = a_ref[...]
        af = a_all.astype(jnp.float32)
        xall = x_ref[...]

        targets = (right, left, diag)
        sends = []
        for j, tgt in enumerate(targets):
            pm = (a_all >> 1) == tgt
            sel = build_sel(pm, ls_T, CAPD)
            sel_store[j] = sel
            xg_send[j] = gather(sel, xall).astype(jnp.bfloat16)
            ag_send[j] = gather(sel, af)
            sx = pltpu.make_async_remote_copy(
                src_ref=xg_send.at[j], dst_ref=xg_recv.at[j],
                send_sem=dx_s.at[j], recv_sem=dx_r.at[j],
                device_id=(tgt,), device_id_type=pl.DeviceIdType.MESH,
            )
            sa = pltpu.make_async_remote_copy(
                src_ref=ag_send.at[j], dst_ref=ag_recv.at[j],
                send_sem=da_s.at[j], recv_sem=da_r.at[j],
                device_id=(tgt,), device_id_type=pl.DeviceIdType.MESH,
            )
            sx.start()
            sa.start()
            sends.append(sx)
            sends.append(sa)

        def wstore(tag, val):
            t, e, c = tag
            if t == "w1":
                w1b[e, :, c * WFC:(c + 1) * WFC] = val
            else:
                w2b[e, c * WFC:(c + 1) * WFC, :] = val

        def load_expert(e):
            jobs = []
            for c in range(F // WFC):
                jobs.append((w1_any.at[e, :, pl.ds(c * WFC, WFC)], ("w1", e, c)))
                jobs.append((w2_any.at[e, pl.ds(c * WFC, WFC), :], ("w2", e, c)))
            copies = []
            for k, (src, tag) in enumerate(jobs):
                cp = pltpu.make_async_copy(src, wstage.at[k % 2], wsem.at[k % 2])
                copies.append(cp)
                cp.start()
                if k >= 1:
                    copies[k - 1].wait()
                    wstore(jobs[k - 1][1],
                           wstage[(k - 1) % 2].astype(jnp.bfloat16))
            copies[-1].wait()
            wstore(jobs[-1][1], wstage[(len(jobs) - 1) % 2].astype(jnp.bfloat16))

        def expert_ffn(e, xblk, mask, ls):
            sel2 = build_sel(mask, ls, CAPE)
            xg = gather(sel2, xblk).astype(jnp.bfloat16)
            h = jnp.maximum(
                jnp.dot(xg, w1b[e], preferred_element_type=jnp.float32), 0.0
            ).astype(jnp.bfloat16)
            o = jnp.dot(
                h, w2b[e], preferred_element_type=jnp.float32
            ).astype(jnp.bfloat16)
            return jnp.dot(sel2, o, preferred_element_type=jnp.float32)

        own = None
        for e in range(E):
            load_expert(e)
            c = expert_ffn(e, xall, a_all == (e0 + e), ls_T)
            own = c if own is None else own + c
        out_ref[...] = own

        sources = (left, right, diag)
        ret_descs = []
        for j, tgt in enumerate(targets):
            rx = pltpu.make_async_remote_copy(
                src_ref=xg_send.at[j], dst_ref=xg_recv.at[j],
                send_sem=dx_s.at[j], recv_sem=dx_r.at[j],
                device_id=(tgt,), device_id_type=pl.DeviceIdType.MESH,
            )
            ra = pltpu.make_async_remote_copy(
                src_ref=ag_send.at[j], dst_ref=ag_recv.at[j],
                send_sem=da_s.at[j], recv_sem=da_r.at[j],
                device_id=(tgt,), device_id_type=pl.DeviceIdType.MESH,
            )
            rx.wait_recv()
            ra.wait_recv()
            xin = xg_recv[j]
            ain = ag_recv[j]
            acc = None
            for e in range(E):
                m2 = ain == (e0 + e).astype(jnp.float32)
                c = expert_ffn(e, xin, m2, ls_D)
                acc = c if acc is None else acc + c
            ret_send[j] = acc.astype(jnp.bfloat16)
            rs = pltpu.make_async_remote_copy(
                src_ref=ret_send.at[j], dst_ref=ret_recv.at[j],
                send_sem=rt_s.at[j], recv_sem=rt_r.at[j],
                device_id=(sources[j],), device_id_type=pl.DeviceIdType.MESH,
            )
            rs.start()
            sends.append(rs)
            ret_descs.append(rs)

        for j, tgt in enumerate(targets):
            ret_descs[j].wait_recv()
            out_ref[...] = out_ref[...] + jnp.dot(
                sel_store[j], ret_recv[j], preferred_element_type=jnp.float32
            )

        for s in sends:
            s.wait_send()

    return pl.pallas_call(
        body,
        out_shape=jax.ShapeDtypeStruct((T, D), jnp.float32),
        in_specs=[
            pl.BlockSpec(memory_space=pltpu.VMEM),
            pl.BlockSpec(memory_space=pltpu.VMEM),
            pl.BlockSpec(memory_space=pltpu.MemorySpace.HBM),
            pl.BlockSpec(memory_space=pltpu.MemorySpace.HBM),
        ],
        out_specs=pl.BlockSpec(memory_space=pltpu.VMEM),
        scratch_shapes=[
            pltpu.VMEM((E, D, F), jnp.bfloat16),
            pltpu.VMEM((E, F, D), jnp.bfloat16),
            pltpu.VMEM((2, D, F // 2), jnp.float32),
            pltpu.VMEM((3, T, CAPD), jnp.bfloat16),
            pltpu.VMEM((3, CAPD, D), jnp.bfloat16),
            pltpu.VMEM((3, CAPD, 1), jnp.float32),
            pltpu.VMEM((3, CAPD, D), jnp.bfloat16),
            pltpu.VMEM((3, CAPD, 1), jnp.float32),
            pltpu.VMEM((3, CAPD, D), jnp.bfloat16),
            pltpu.VMEM((3, CAPD, D), jnp.bfloat16),
            pltpu.SemaphoreType.DMA((2,)),
            pltpu.SemaphoreType.DMA((3,)),
            pltpu.SemaphoreType.DMA((3,)),
            pltpu.SemaphoreType.DMA((3,)),
            pltpu.SemaphoreType.DMA((3,)),
            pltpu.SemaphoreType.DMA((3,)),
            pltpu.SemaphoreType.DMA((3,)),
        ],
        compiler_params=pltpu.CompilerParams(
            collective_id=0,
            vmem_limit_bytes=60 * 1024 * 1024,
        ),
    )(xb, a2d, W1, W2)
